# baseline (speedup 1.0000x reference)
import functools

import jax
import jax.numpy as jnp
from jax import lax
from jax.experimental import pallas as pl
from jax.experimental.pallas import tpu as pltpu
from jax.experimental.pallas import tpu_sc as plsc


def _mlp_table_body(tab_ref, w1_ref, b1_ref, w2_ref, b2_ref, y_ref, ybf_ref):
    h = jnp.dot(tab_ref[...], w1_ref[...], preferred_element_type=jnp.float32)
    h = jnp.maximum(h + b1_ref[...], 0.0)
    y = jnp.dot(h, w2_ref[...], preferred_element_type=jnp.float32) + b2_ref[...]
    y_ref[...] = y
    ybf_ref[...] = y.astype(jnp.bfloat16)


def _compute_vocab_outputs(embed_table, W1, b1, W2, b2):
    V = embed_table.shape[0]
    H = W1.shape[1]
    O = W2.shape[1]
    return pl.pallas_call(
        _mlp_table_body,
        out_shape=[
            jax.ShapeDtypeStruct((V, O), jnp.float32),
            jax.ShapeDtypeStruct((V, O), jnp.bfloat16),
        ],
    )(embed_table, W1, b1.reshape(1, H), W2, b2.reshape(1, O))


@functools.cache
def _make_gather(V, D, N):
    info = plsc.get_sparse_core_info()
    NC, NS = info.num_cores, info.num_subcores
    NW = NC * NS
    assert N % NW == 0
    n_per = N // NW
    NBUF = 2
    C = 200
    assert n_per % (NBUF * C) == 0
    rounds = n_per // (NBUF * C)

    mesh = plsc.VectorSubcoreMesh(core_axis_name="c", subcore_axis_name="s")

    @functools.partial(
        pl.kernel,
        out_type=jax.ShapeDtypeStruct((N, D), jnp.float32),
        mesh=mesh,
        scratch_types=[
            pltpu.VMEM((n_per,), jnp.int32),
        ]
        + [pltpu.VMEM((C, D), jnp.float32)] * NBUF
        + [pltpu.SemaphoreType.DMA] * (2 * NBUF),
    )
    def gather(y_hbm, idx_hbm, out_hbm, idx_v, *bufs_and_sems):
        rows = bufs_and_sems[:NBUF]
        gsem = bufs_and_sems[NBUF : 2 * NBUF]
        ssem = bufs_and_sems[2 * NBUF :]
        wid = lax.axis_index("s") * NC + lax.axis_index("c")
        base = wid * n_per
        pltpu.sync_copy(idx_hbm.at[pl.ds(base, n_per)], idx_v)

        def start_gather(g, buf, sem):
            pltpu.async_copy(y_hbm.at[idx_v.at[pl.ds(g * C, C)]], buf, sem)

        def wait_gather(buf, sem):
            pltpu.make_async_copy(y_hbm.at[idx_v.at[pl.ds(0, C)]], buf, sem).wait()

        n_chunks = rounds * NBUF
        start_gather(0, rows[0], gsem[0])
        start_gather(1, rows[1], gsem[1])

        def body(i, carry):
            g0 = 2 * i

            wait_gather(rows[0], gsem[0])
            pltpu.sync_copy(rows[0], out_hbm.at[pl.ds(base + g0 * C, C)])

            @pl.when(g0 + 2 < n_chunks)
            def _():
                start_gather(g0 + 2, rows[0], gsem[0])

            wait_gather(rows[1], gsem[1])
            pltpu.sync_copy(rows[1], out_hbm.at[pl.ds(base + (g0 + 1) * C, C)])

            @pl.when(g0 + 3 < n_chunks)
            def _():
                start_gather(g0 + 3, rows[1], gsem[1])

            return carry

        lax.fori_loop(0, n_chunks // 2, body, 0)

    return gather


def _onehot_body(ids_ref, y_ref, out_ref):
    ids_blk = ids_ref[...]
    iota = lax.broadcasted_iota(
        jnp.int32, (ids_blk.shape[0], y_ref.shape[0]), 1
    )
    oh = (ids_blk == iota).astype(jnp.bfloat16)
    out_ref[...] = jnp.dot(oh, y_ref[...], preferred_element_type=jnp.float32)


def _tc_onehot_lookup(y_bf16, idx, Bt=512):
    N = idx.shape[0]
    Vp, D = y_bf16.shape
    assert N % Bt == 0
    return pl.pallas_call(
        _onehot_body,
        grid=(N // Bt,),
        in_specs=[
            pl.BlockSpec((Bt, 1), lambda i: (i, 0)),
            pl.BlockSpec((Vp, D), lambda i: (0, 0)),
        ],
        out_specs=pl.BlockSpec((Bt, D), lambda i: (i, 0)),
        out_shape=jax.ShapeDtypeStruct((N, D), jnp.float32),
    )(idx.reshape(N, 1), y_bf16)


def kernel(ids, embed_table, W1, b1, W2, b2):
    B, L = ids.shape
    V = embed_table.shape[0]
    O = W2.shape[1]
    Vp = 1024
    tab_pad = jnp.concatenate(
        [embed_table, jnp.zeros((Vp - V, embed_table.shape[1]), jnp.float32)]
    )
    y, y_bf16 = _compute_vocab_outputs(tab_pad, W1, b1, W2, b2)
    idx = ids.reshape(-1).astype(jnp.int32)
    N = B * L
    N_SC = 128000
    out_sc = _make_gather(Vp, O, N_SC)(y, idx[:N_SC])
    out_tc = _tc_onehot_lookup(y_bf16, idx[N_SC:])
    out = jnp.concatenate([out_sc, out_tc], axis=0)
    return out.reshape(B, L, O)


# speedup vs baseline: 1.8980x; 1.8980x over previous
import functools

import jax
import jax.numpy as jnp
from jax import lax
from jax.experimental import pallas as pl
from jax.experimental.pallas import tpu as pltpu
from jax.experimental.pallas import tpu_sc as plsc


def _mlp_table_body(tab_ref, w1_ref, b1_ref, w2_ref, b2_ref, y_ref):
    h = jnp.dot(tab_ref[...], w1_ref[...], preferred_element_type=jnp.float32)
    h = jnp.maximum(h + b1_ref[...], 0.0)
    y_ref[...] = (
        jnp.dot(h, w2_ref[...], preferred_element_type=jnp.float32) + b2_ref[...]
    )


def _compute_vocab_outputs(embed_table, W1, b1, W2, b2):
    V = embed_table.shape[0]
    H = W1.shape[1]
    O = W2.shape[1]
    return pl.pallas_call(
        _mlp_table_body,
        out_shape=jax.ShapeDtypeStruct((V, O), jnp.float32),
    )(embed_table, W1, b1.reshape(1, H), W2, b2.reshape(1, O))


@functools.cache
def _make_gather(V, D, N):
    info = plsc.get_sparse_core_info()
    NC, NS = info.num_cores, info.num_subcores
    NW = NC * NS
    assert N % NW == 0
    n_per = N // NW
    NBUF = 2
    C = 200
    assert n_per % (NBUF * C) == 0
    rounds = n_per // (NBUF * C)

    mesh = plsc.VectorSubcoreMesh(core_axis_name="c", subcore_axis_name="s")

    @functools.partial(
        pl.kernel,
        out_type=jax.ShapeDtypeStruct((N, D), jnp.float32),
        mesh=mesh,
        scratch_types=[
            pltpu.VMEM((n_per,), jnp.int32),
        ]
        + [pltpu.VMEM((C, D), jnp.float32)] * NBUF
        + [pltpu.SemaphoreType.DMA] * (2 * NBUF),
    )
    def gather(y_hbm, idx_hbm, out_hbm, idx_v, *bufs_and_sems):
        rows = bufs_and_sems[:NBUF]
        gsem = bufs_and_sems[NBUF : 2 * NBUF]
        ssem = bufs_and_sems[2 * NBUF :]
        wid = lax.axis_index("s") * NC + lax.axis_index("c")
        base = wid * n_per
        pltpu.sync_copy(idx_hbm.at[pl.ds(base, n_per)], idx_v)

        def start_gather(g, buf, sem):
            pltpu.async_copy(y_hbm.at[idx_v.at[pl.ds(g * C, C)]], buf, sem)

        def wait_gather(buf, sem):
            pltpu.make_async_copy(y_hbm.at[idx_v.at[pl.ds(0, C)]], buf, sem).wait()

        n_chunks = rounds * NBUF
        start_gather(0, rows[0], gsem[0])
        start_gather(1, rows[1], gsem[1])

        def body(i, carry):
            g0 = 2 * i

            wait_gather(rows[0], gsem[0])
            pltpu.sync_copy(rows[0], out_hbm.at[pl.ds(base + g0 * C, C)])

            @pl.when(g0 + 2 < n_chunks)
            def _():
                start_gather(g0 + 2, rows[0], gsem[0])

            wait_gather(rows[1], gsem[1])
            pltpu.sync_copy(rows[1], out_hbm.at[pl.ds(base + (g0 + 1) * C, C)])

            @pl.when(g0 + 3 < n_chunks)
            def _():
                start_gather(g0 + 3, rows[1], gsem[1])

            return carry

        lax.fori_loop(0, n_chunks // 2, body, 0)

    return gather


def kernel(ids, embed_table, W1, b1, W2, b2):
    B, L = ids.shape
    V = embed_table.shape[0]
    O = W2.shape[1]
    y = _compute_vocab_outputs(embed_table, W1, b1, W2, b2)
    idx = ids.reshape(-1).astype(jnp.int32)
    out = _make_gather(V, O, B * L)(y, idx)
    return out.reshape(B, L, O)
